# baseline (speedup 1.0000x reference)
import functools

import jax
import jax.numpy as jnp
from jax import lax
from jax.experimental import pallas as pl
from jax.experimental.pallas import tpu as pltpu
from jax.experimental.pallas import tpu_sc as plsc

N = 10000
E = 320000
FEA = 128
HID = 128
TGT = 16
G = 64

NC = 2
NS = 16
NW = NC * NS
EPW = E // NW
RING = 6
CH = 40
NCHUNK = EPW // CH
CPB = 24
NBLKI = (NCHUNK + CPB - 1) // CPB
NPAD = 10240
RPW = NPAD // NS
ZR = 8


def _zero_issue(sid, zero_v, agg_sh, sem):
    z16 = jnp.zeros((16,), jnp.float32)

    def zrow(i, carry):
        for j in range(FEA // 16):
            zero_v[i, pl.ds(j * 16, 16)] = z16
        return carry

    lax.fori_loop(0, ZR, zrow, 0)

    def zcopy(i, carry):
        pltpu.async_copy(zero_v, agg_sh.at[pl.ds(sid * RPW + i * ZR, ZR)],
                         sem)
        return carry

    lax.fori_loop(0, RPW // ZR, zcopy, 0)


def _zero_drain(sid, zero_v, agg_sh, sem):
    def zdrain(i, carry):
        pltpu.make_async_copy(
            zero_v, agg_sh.at[pl.ds(sid * RPW + i * ZR, ZR)], sem).wait()
        return carry

    lax.fori_loop(0, RPW // ZR, zdrain, 0)


def _writeout(cid, sid, agg_sh, out_hbm):
    @pl.when(sid < NS - 1)
    def _():
        pltpu.sync_copy(agg_sh.at[pl.ds(sid * RPW, RPW)],
                        out_hbm.at[pl.ds(cid * N + sid * RPW, RPW)])

    @pl.when(sid == NS - 1)
    def _():
        pltpu.sync_copy(agg_sh.at[pl.ds((NS - 1) * RPW, N - (NS - 1) * RPW)],
                        out_hbm.at[pl.ds(cid * N + (NS - 1) * RPW,
                                         N - (NS - 1) * RPW)])


def _stage_and_prime(h_hbm, src_hbm, dst_hbm, row, b, src_v, dst_v,
                     bufs, semg):
    pltpu.sync_copy(src_hbm.at[row, pl.ds(b * CPB, CPB)], src_v)
    pltpu.sync_copy(dst_hbm.at[row, pl.ds(b * CPB, CPB)], dst_v)
    for r in range(RING - 1):
        pltpu.async_copy(h_hbm.at[src_v.at[r]], bufs[r], semg[r])


def _edge_pipeline(h_hbm, src_hbm, dst_hbm, row, nchunk, nblki,
                   src_v, dst_v, bufs, semg, sems, agg_sh):
    def block(b, carry):
        base_c = b * CPB

        @pl.when(b > 0)
        def _():
            _stage_and_prime(h_hbm, src_hbm, dst_hbm, row, b, src_v, dst_v,
                             bufs, semg)

        def group(g, carry2):
            for j in range(RING):
                c = RING * g + j
                cp = c + RING - 1
                jp = (j + RING - 1) % RING

                @pl.when((cp < CPB) & (cp >= RING)
                         & (base_c + cp - RING < nchunk))
                def _():
                    pltpu.make_async_copy(
                        bufs[jp], agg_sh.at[dst_v.at[cp - RING]],
                        sems[jp]).wait()

                @pl.when((cp < CPB) & (base_c + cp < nchunk))
                def _():
                    pltpu.async_copy(h_hbm.at[src_v.at[cp]], bufs[jp],
                                     semg[jp])

                @pl.when(base_c + c < nchunk)
                def _():
                    pltpu.make_async_copy(h_hbm.at[src_v.at[c]], bufs[j],
                                          semg[j]).wait()
                    pltpu.async_copy(bufs[j], agg_sh.at[dst_v.at[c]],
                                     sems[j], add=True)

            return carry2

        lax.fori_loop(0, CPB // RING, group, 0)

        for t in range(CPB - RING, CPB):
            @pl.when(base_c + t < nchunk)
            def _():
                pltpu.make_async_copy(bufs[t % RING],
                                      agg_sh.at[dst_v.at[t]],
                                      sems[t % RING]).wait()

        return carry

    lax.fori_loop(0, nblki, block, 0)


def _split_scratch(scr):
    src_v, dst_v = scr[0], scr[1]
    bufs = scr[2:2 + RING]
    zero_v = scr[2 + RING]
    agg_sh = scr[3 + RING]
    semg = scr[4 + RING:4 + 2 * RING]
    sems = scr[4 + 2 * RING:4 + 3 * RING]
    return src_v, dst_v, bufs, zero_v, agg_sh, semg, sems


def _sc_agg_body(h_hbm, src_hbm, dst_hbm, out_hbm, *scr):
    cid = lax.axis_index("c")
    sid = lax.axis_index("s")
    wid = cid * NS + sid
    src_v, dst_v, bufs, zero_v, agg_sh, semg, sems = _split_scratch(scr)
    _zero_issue(sid, zero_v, agg_sh, sems[0])
    _stage_and_prime(h_hbm, src_hbm, dst_hbm, wid, 0, src_v, dst_v,
                     bufs, semg)
    _zero_drain(sid, zero_v, agg_sh, sems[0])
    plsc.subcore_barrier()
    _edge_pipeline(h_hbm, src_hbm, dst_hbm, wid, NCHUNK, NBLKI,
                   src_v, dst_v, bufs, semg, sems, agg_sh)
    plsc.subcore_barrier()
    _writeout(cid, sid, agg_sh, out_hbm)


NCHUNK2 = 2 * NCHUNK
NBLKI2 = (NCHUNK2 + CPB - 1) // CPB


def _sc_agg2_body(hd_hbm, ha_hbm, src_hbm, dst_hbm, out_hbm, *scr):
    cid = lax.axis_index("c")
    sid = lax.axis_index("s")
    src_v, dst_v, bufs, zero_v, agg_sh, semg, sems = _split_scratch(scr)
    _zero_issue(sid, zero_v, agg_sh, sems[0])

    @pl.when(cid == 0)
    def _():
        _stage_and_prime(hd_hbm, src_hbm, dst_hbm, sid, 0, src_v, dst_v,
                         bufs, semg)

    @pl.when(cid == 1)
    def _():
        _stage_and_prime(ha_hbm, src_hbm, dst_hbm, sid, 0, src_v, dst_v,
                         bufs, semg)

    _zero_drain(sid, zero_v, agg_sh, sems[0])
    plsc.subcore_barrier()
    args = (src_hbm, dst_hbm, sid, NCHUNK2, NBLKI2, src_v, dst_v,
            bufs, semg, sems, agg_sh)

    @pl.when(cid == 0)
    def _():
        _edge_pipeline(hd_hbm, *args)

    @pl.when(cid == 1)
    def _():
        _edge_pipeline(ha_hbm, *args)

    plsc.subcore_barrier()
    _writeout(cid, sid, agg_sh, out_hbm)


_SC_SCRATCH = (
    [pltpu.VMEM((CPB, CH), jnp.int32),
     pltpu.VMEM((CPB, CH), jnp.int32)]
    + [pltpu.VMEM((CH, FEA), jnp.float32) for _ in range(RING)]
    + [pltpu.VMEM((ZR, FEA), jnp.float32),
       pltpu.VMEM_SHARED((NPAD, FEA), jnp.float32)]
    + [pltpu.SemaphoreType.DMA for _ in range(2 * RING)]
)


@functools.cache
def _get_sc_agg():
    return pl.kernel(
        _sc_agg_body,
        mesh=plsc.VectorSubcoreMesh(core_axis_name="c", subcore_axis_name="s"),
        out_type=jax.ShapeDtypeStruct((NC * N, FEA), jnp.float32),
        scratch_types=list(_SC_SCRATCH),
    )


@functools.cache
def _get_sc_agg2():
    return pl.kernel(
        _sc_agg2_body,
        mesh=plsc.VectorSubcoreMesh(core_axis_name="c", subcore_axis_name="s"),
        out_type=jax.ShapeDtypeStruct((NC * N, FEA), jnp.float32),
        scratch_types=list(_SC_SCRATCH),
    )

BLK = 1000
NBLK = N // BLK
_PREC = jax.lax.Precision.DEFAULT


def _mm(a, b):
    return jnp.dot(a, b, precision=_PREC, preferred_element_type=jnp.float32)


def _l0_body(x_ref, p0_ref, p1_ref,
             w1d_ref, b1d_ref, w2d_ref, b2d_ref,
             w1a_ref, b1a_ref, w2a_ref, b2a_ref,
             hd_ref, ha_ref):
    m = x_ref[...] + p0_ref[...] + p1_ref[...]
    md = m[:, FEA - 1:FEA]
    td = jnp.maximum(md * w1d_ref[...] + b1d_ref[...], 0.0)
    hd_ref[...] = jnp.maximum(_mm(td, w2d_ref[...]) + b2d_ref[...], 0.0)
    ta = jnp.maximum(_mm(m, w1a_ref[...]) + b1a_ref[...], 0.0)
    ha_ref[...] = jnp.maximum(_mm(ta, w2a_ref[...]) + b2a_ref[...], 0.0)


def _full(shape):
    return pl.BlockSpec(shape, lambda i: (0, 0))


def _rows(shape):
    return pl.BlockSpec(shape, lambda i: (i, 0))


_l0 = pl.pallas_call(
    _l0_body,
    grid=(NBLK,),
    in_specs=[
        _rows((BLK, FEA)),
        _rows((BLK, FEA)),
        pl.BlockSpec((BLK, FEA), lambda i: (i + NBLK, 0)),
        _full((1, HID)), _full((1, HID)), _full((HID, HID)), _full((1, HID)),
        _full((FEA, HID)), _full((1, HID)), _full((HID, HID)), _full((1, HID)),
    ],
    out_specs=[_rows((BLK, HID)), _rows((BLK, HID))],
    out_shape=[jax.ShapeDtypeStruct((N, HID), jnp.float32),
               jax.ShapeDtypeStruct((N, HID), jnp.float32)],
)


def _log_softmax(o):
    mx = jnp.max(o, axis=-1, keepdims=True)
    return o - (jnp.log(jnp.sum(jnp.exp(o - mx), axis=-1, keepdims=True)) + mx)


def _l1_body(hd_ref, pd_ref, ha_ref, pa_ref, b3_ref,
             w1d_ref, b1d_ref, w2d_ref, b2d_ref,
             w1a_ref, b1a_ref, w2a_ref, b2a_ref,
             wod_ref, bod_ref, woa_ref, boa_ref, al_ref,
             out_ref, poold_ref, poola_ref):
    i = pl.program_id(0)
    md = hd_ref[...] + pd_ref[...]
    td = jnp.maximum(_mm(md, w1d_ref[...]) + b1d_ref[...], 0.0)
    h2d = jnp.maximum(_mm(td, w2d_ref[...]) + b2d_ref[...], 0.0)
    ma = ha_ref[...] + pa_ref[...]
    ta = jnp.maximum(_mm(ma, w1a_ref[...]) + b1a_ref[...], 0.0)
    h2a = jnp.maximum(_mm(ta, w2a_ref[...]) + b2a_ref[...], 0.0)

    bvec = b3_ref[0]
    oh = (lax.broadcasted_iota(jnp.int32, (G, BLK), 0)
          == jnp.broadcast_to(bvec, (G, BLK))).astype(jnp.float32)

    @pl.when(i == 0)
    def _():
        poold_ref[...] = jnp.zeros_like(poold_ref)
        poola_ref[...] = jnp.zeros_like(poola_ref)

    poold_ref[...] += _mm(oh, h2d)
    poola_ref[...] += _mm(oh, h2a)

    @pl.when(i == NBLK - 1)
    def _():
        o1 = _mm(poold_ref[...], wod_ref[...]) + bod_ref[...]
        o2 = _mm(poola_ref[...], woa_ref[...]) + boa_ref[...]
        a = al_ref[0]
        out_ref[...] = a * _log_softmax(o1) + (1.0 - a) * _log_softmax(o2)


_l1 = pl.pallas_call(
    _l1_body,
    grid=(NBLK,),
    in_specs=[
        _rows((BLK, HID)),
        _rows((BLK, HID)),
        _rows((BLK, HID)),
        pl.BlockSpec((BLK, HID), lambda i: (i + NBLK, 0)),
        pl.BlockSpec((1, 1, BLK), lambda i: (i, 0, 0)),
        _full((HID, HID)), _full((1, HID)), _full((HID, HID)), _full((1, HID)),
        _full((HID, HID)), _full((1, HID)), _full((HID, HID)), _full((1, HID)),
        _full((HID, TGT)), _full((1, TGT)), _full((HID, TGT)),
        _full((1, TGT)),
        pl.BlockSpec(memory_space=pltpu.SMEM),
    ],
    out_specs=pl.BlockSpec((G, TGT), lambda i: (0, 0)),
    out_shape=jax.ShapeDtypeStruct((G, TGT), jnp.float32),
    scratch_shapes=[pltpu.VMEM((G, HID), jnp.float32),
                    pltpu.VMEM((G, HID), jnp.float32)],
)


def kernel(x, edge_index, batch, params):
    p = params
    pad3 = ((0, 0), (0, NBLKI * CPB - NCHUNK), (0, 0))
    src = jnp.pad(edge_index[0].reshape(NW, NCHUNK, CH), pad3)
    dst = jnp.pad(edge_index[1].reshape(NW, NCHUNK, CH), pad3)

    _sc_agg = _get_sc_agg()
    parts_x = _sc_agg(x, src, dst)

    w1a0 = jnp.pad(p['ga_W1_0'], ((0, 1), (0, 0)))

    def r2(v):
        return v.reshape(1, -1)

    hd, ha = _l0(
        x, parts_x, parts_x,
        p['gd_W1_0'], r2(p['gd_b1_0']), p['gd_W2_0'], r2(p['gd_b2_0']),
        w1a0, r2(p['ga_b1_0']), p['ga_W2_0'], r2(p['ga_b2_0']),
    )

    src2 = jnp.pad(edge_index[0].reshape(NS, NCHUNK2, CH),
                   ((0, 0), (0, NBLKI2 * CPB - NCHUNK2), (0, 0)))
    dst2 = jnp.pad(edge_index[1].reshape(NS, NCHUNK2, CH),
                   ((0, 0), (0, NBLKI2 * CPB - NCHUNK2), (0, 0)))
    parts2 = _get_sc_agg2()(hd, ha, src2, dst2)

    batch3 = batch.reshape(NBLK, 1, BLK)
    alpha = jax.nn.sigmoid(p['alpha']).reshape(1)
    out = _l1(
        hd, parts2, ha, parts2, batch3,
        p['gd_W1_1'], r2(p['gd_b1_1']), p['gd_W2_1'], r2(p['gd_b2_1']),
        p['ga_W1_1'], r2(p['ga_b1_1']), p['ga_W2_1'], r2(p['ga_b2_1']),
        p['gd_Wo'], r2(p['gd_bo']), p['ga_Wo'], r2(p['ga_bo']),
        alpha,
    )
    return out


# speedup vs baseline: 1.1180x; 1.1180x over previous
import functools

import jax
import jax.numpy as jnp
from jax import lax
from jax.experimental import pallas as pl
from jax.experimental.pallas import tpu as pltpu
from jax.experimental.pallas import tpu_sc as plsc

N = 10000
E = 320000
FEA = 128
HID = 128
TGT = 16
G = 64

NC = 2
NS = 16
NW = NC * NS
EPW = E // NW
CH = 80
NCHUNK = EPW // CH
CPB = 24
NBLKI = 6
NPAD = 10240
RPW = NPAD // NS
ZR = 8


def _zero_issue(sid, zero_v, agg_sh, sem):
    z16 = jnp.zeros((16,), jnp.float32)

    def zrow(i, carry):
        for j in range(FEA // 16):
            zero_v[i, pl.ds(j * 16, 16)] = z16
        return carry

    lax.fori_loop(0, ZR, zrow, 0)

    def zcopy(i, carry):
        pltpu.async_copy(zero_v, agg_sh.at[pl.ds(sid * RPW + i * ZR, ZR)],
                         sem)
        return carry

    lax.fori_loop(0, RPW // ZR, zcopy, 0)


def _zero_drain(sid, zero_v, agg_sh, sem):
    def zdrain(i, carry):
        pltpu.make_async_copy(
            zero_v, agg_sh.at[pl.ds(sid * RPW + i * ZR, ZR)], sem).wait()
        return carry

    lax.fori_loop(0, RPW // ZR, zdrain, 0)


def _writeout(cid, sid, agg_sh, out_hbm):
    @pl.when(sid < NS - 1)
    def _():
        pltpu.sync_copy(agg_sh.at[pl.ds(sid * RPW, RPW)],
                        out_hbm.at[pl.ds(cid * N + sid * RPW, RPW)])

    @pl.when(sid == NS - 1)
    def _():
        pltpu.sync_copy(agg_sh.at[pl.ds((NS - 1) * RPW, N - (NS - 1) * RPW)],
                        out_hbm.at[pl.ds(cid * N + (NS - 1) * RPW,
                                         N - (NS - 1) * RPW)])


def _stage_and_prime(h_hbm, src_hbm, dst_hbm, row, b, src_v, dst_v,
                     bufs, semg):
    pltpu.sync_copy(src_hbm.at[row, pl.ds(b * CPB, CPB)], src_v)
    pltpu.sync_copy(dst_hbm.at[row, pl.ds(b * CPB, CPB)], dst_v)
    pltpu.async_copy(h_hbm.at[src_v.at[0]], bufs[0], semg[0])
    pltpu.async_copy(h_hbm.at[src_v.at[1]], bufs[1], semg[1])


def _edge_pipeline(h_hbm, src_hbm, dst_hbm, row, nchunk, nblki,
                   src_v, dst_v, bufs, semg, sems, agg_sh):
    def block(b, carry):
        base_c = b * CPB

        @pl.when(b > 0)
        def _():
            _stage_and_prime(h_hbm, src_hbm, dst_hbm, row, b, src_v, dst_v,
                             bufs, semg)

        def group(g, carry2):
            for j in range(3):
                c = 3 * g + j
                cp = c + 2
                jp = (j + 2) % 3

                @pl.when((cp < CPB) & (cp >= 3)
                         & (base_c + cp - 3 < nchunk))
                def _():
                    pltpu.make_async_copy(
                        bufs[jp], agg_sh.at[dst_v.at[cp - 3]],
                        sems[jp]).wait()

                @pl.when((cp < CPB) & (base_c + cp < nchunk))
                def _():
                    pltpu.async_copy(h_hbm.at[src_v.at[cp]], bufs[jp],
                                     semg[jp])

                @pl.when(base_c + c < nchunk)
                def _():
                    pltpu.make_async_copy(h_hbm.at[src_v.at[c]], bufs[j],
                                          semg[j]).wait()
                    pltpu.async_copy(bufs[j], agg_sh.at[dst_v.at[c]],
                                     sems[j], add=True)

            return carry2

        lax.fori_loop(0, CPB // 3, group, 0)

        for t in range(CPB - 3, CPB):
            @pl.when(base_c + t < nchunk)
            def _():
                pltpu.make_async_copy(bufs[t % 3],
                                      agg_sh.at[dst_v.at[t]],
                                      sems[t % 3]).wait()

        return carry

    lax.fori_loop(0, nblki, block, 0)


def _sc_agg_body(h_hbm, src_hbm, dst_hbm, out_hbm,
                 src_v, dst_v, buf0_v, buf1_v, buf2_v, zero_v, agg_sh,
                 semg0, semg1, semg2, sems0, sems1, sems2):
    cid = lax.axis_index("c")
    sid = lax.axis_index("s")
    wid = cid * NS + sid
    bufs = (buf0_v, buf1_v, buf2_v)
    semg = (semg0, semg1, semg2)
    _zero_issue(sid, zero_v, agg_sh, sems0)
    _stage_and_prime(h_hbm, src_hbm, dst_hbm, wid, 0, src_v, dst_v,
                     bufs, semg)
    _zero_drain(sid, zero_v, agg_sh, sems0)
    plsc.subcore_barrier()
    _edge_pipeline(h_hbm, src_hbm, dst_hbm, wid, NCHUNK, NBLKI,
                   src_v, dst_v, bufs, semg, (sems0, sems1, sems2), agg_sh)
    plsc.subcore_barrier()
    _writeout(cid, sid, agg_sh, out_hbm)


NCHUNK2 = 2 * NCHUNK
NBLKI2 = (NCHUNK2 + CPB - 1) // CPB


def _sc_agg2_body(hd_hbm, ha_hbm, src_hbm, dst_hbm, out_hbm,
                  src_v, dst_v, buf0_v, buf1_v, buf2_v, zero_v, agg_sh,
                  semg0, semg1, semg2, sems0, sems1, sems2):
    cid = lax.axis_index("c")
    sid = lax.axis_index("s")
    bufs = (buf0_v, buf1_v, buf2_v)
    semg = (semg0, semg1, semg2)
    _zero_issue(sid, zero_v, agg_sh, sems0)

    @pl.when(cid == 0)
    def _():
        _stage_and_prime(hd_hbm, src_hbm, dst_hbm, sid, 0, src_v, dst_v,
                         bufs, semg)

    @pl.when(cid == 1)
    def _():
        _stage_and_prime(ha_hbm, src_hbm, dst_hbm, sid, 0, src_v, dst_v,
                         bufs, semg)

    _zero_drain(sid, zero_v, agg_sh, sems0)
    plsc.subcore_barrier()
    args = (src_hbm, dst_hbm, sid, NCHUNK2, NBLKI2, src_v, dst_v,
            bufs, semg, (sems0, sems1, sems2), agg_sh)

    @pl.when(cid == 0)
    def _():
        _edge_pipeline(hd_hbm, *args)

    @pl.when(cid == 1)
    def _():
        _edge_pipeline(ha_hbm, *args)

    plsc.subcore_barrier()
    _writeout(cid, sid, agg_sh, out_hbm)


_SC_SCRATCH = [
    pltpu.VMEM((CPB, CH), jnp.int32),
    pltpu.VMEM((CPB, CH), jnp.int32),
    pltpu.VMEM((CH, FEA), jnp.float32),
    pltpu.VMEM((CH, FEA), jnp.float32),
    pltpu.VMEM((CH, FEA), jnp.float32),
    pltpu.VMEM((ZR, FEA), jnp.float32),
    pltpu.VMEM_SHARED((NPAD, FEA), jnp.float32),
    pltpu.SemaphoreType.DMA,
    pltpu.SemaphoreType.DMA,
    pltpu.SemaphoreType.DMA,
    pltpu.SemaphoreType.DMA,
    pltpu.SemaphoreType.DMA,
    pltpu.SemaphoreType.DMA,
]


@functools.cache
def _get_sc_agg():
    return pl.kernel(
        _sc_agg_body,
        mesh=plsc.VectorSubcoreMesh(core_axis_name="c", subcore_axis_name="s"),
        out_type=jax.ShapeDtypeStruct((NC * N, FEA), jnp.float32),
        scratch_types=list(_SC_SCRATCH),
    )


@functools.cache
def _get_sc_agg2():
    return pl.kernel(
        _sc_agg2_body,
        mesh=plsc.VectorSubcoreMesh(core_axis_name="c", subcore_axis_name="s"),
        out_type=jax.ShapeDtypeStruct((NC * N, FEA), jnp.float32),
        scratch_types=list(_SC_SCRATCH),
    )

BLK = 1000
NBLK = N // BLK
_PREC = jax.lax.Precision.DEFAULT


def _mm(a, b):
    return jnp.dot(a, b, precision=_PREC, preferred_element_type=jnp.float32)


def _l0_body(x_ref, p0_ref, p1_ref,
             w1d_ref, b1d_ref, w2d_ref, b2d_ref,
             w1a_ref, b1a_ref, w2a_ref, b2a_ref,
             hd_ref, ha_ref):
    m = x_ref[...] + p0_ref[...] + p1_ref[...]
    md = m[:, FEA - 1:FEA]
    td = jnp.maximum(md * w1d_ref[...] + b1d_ref[...], 0.0)
    hd_ref[...] = jnp.maximum(_mm(td, w2d_ref[...]) + b2d_ref[...], 0.0)
    ta = jnp.maximum(_mm(m, w1a_ref[...]) + b1a_ref[...], 0.0)
    ha_ref[...] = jnp.maximum(_mm(ta, w2a_ref[...]) + b2a_ref[...], 0.0)


def _full(shape):
    return pl.BlockSpec(shape, lambda i: (0, 0))


def _rows(shape):
    return pl.BlockSpec(shape, lambda i: (i, 0))


_l0 = pl.pallas_call(
    _l0_body,
    grid=(NBLK,),
    in_specs=[
        _rows((BLK, FEA)),
        _rows((BLK, FEA)),
        pl.BlockSpec((BLK, FEA), lambda i: (i + NBLK, 0)),
        _full((1, HID)), _full((1, HID)), _full((HID, HID)), _full((1, HID)),
        _full((FEA, HID)), _full((1, HID)), _full((HID, HID)), _full((1, HID)),
    ],
    out_specs=[_rows((BLK, HID)), _rows((BLK, HID))],
    out_shape=[jax.ShapeDtypeStruct((N, HID), jnp.float32),
               jax.ShapeDtypeStruct((N, HID), jnp.float32)],
)


def _log_softmax(o):
    mx = jnp.max(o, axis=-1, keepdims=True)
    return o - (jnp.log(jnp.sum(jnp.exp(o - mx), axis=-1, keepdims=True)) + mx)


def _l1_body(hd_ref, pd_ref, ha_ref, pa_ref, b3_ref,
             w1d_ref, b1d_ref, w2d_ref, b2d_ref,
             w1a_ref, b1a_ref, w2a_ref, b2a_ref,
             wod_ref, bod_ref, woa_ref, boa_ref, al_ref,
             out_ref, poold_ref, poola_ref):
    i = pl.program_id(0)
    md = hd_ref[...] + pd_ref[...]
    td = jnp.maximum(_mm(md, w1d_ref[...]) + b1d_ref[...], 0.0)
    h2d = jnp.maximum(_mm(td, w2d_ref[...]) + b2d_ref[...], 0.0)
    ma = ha_ref[...] + pa_ref[...]
    ta = jnp.maximum(_mm(ma, w1a_ref[...]) + b1a_ref[...], 0.0)
    h2a = jnp.maximum(_mm(ta, w2a_ref[...]) + b2a_ref[...], 0.0)

    bvec = b3_ref[0]
    oh = (lax.broadcasted_iota(jnp.int32, (G, BLK), 0)
          == jnp.broadcast_to(bvec, (G, BLK))).astype(jnp.float32)

    @pl.when(i == 0)
    def _():
        poold_ref[...] = jnp.zeros_like(poold_ref)
        poola_ref[...] = jnp.zeros_like(poola_ref)

    poold_ref[...] += _mm(oh, h2d)
    poola_ref[...] += _mm(oh, h2a)

    @pl.when(i == NBLK - 1)
    def _():
        o1 = _mm(poold_ref[...], wod_ref[...]) + bod_ref[...]
        o2 = _mm(poola_ref[...], woa_ref[...]) + boa_ref[...]
        a = al_ref[0]
        out_ref[...] = a * _log_softmax(o1) + (1.0 - a) * _log_softmax(o2)


_l1 = pl.pallas_call(
    _l1_body,
    grid=(NBLK,),
    in_specs=[
        _rows((BLK, HID)),
        _rows((BLK, HID)),
        _rows((BLK, HID)),
        pl.BlockSpec((BLK, HID), lambda i: (i + NBLK, 0)),
        pl.BlockSpec((1, 1, BLK), lambda i: (i, 0, 0)),
        _full((HID, HID)), _full((1, HID)), _full((HID, HID)), _full((1, HID)),
        _full((HID, HID)), _full((1, HID)), _full((HID, HID)), _full((1, HID)),
        _full((HID, TGT)), _full((1, TGT)), _full((HID, TGT)),
        _full((1, TGT)),
        pl.BlockSpec(memory_space=pltpu.SMEM),
    ],
    out_specs=pl.BlockSpec((G, TGT), lambda i: (0, 0)),
    out_shape=jax.ShapeDtypeStruct((G, TGT), jnp.float32),
    scratch_shapes=[pltpu.VMEM((G, HID), jnp.float32),
                    pltpu.VMEM((G, HID), jnp.float32)],
)


def kernel(x, edge_index, batch, params):
    p = params
    pad3 = ((0, 0), (0, NBLKI * CPB - NCHUNK), (0, 0))
    src = jnp.pad(edge_index[0].reshape(NW, NCHUNK, CH), pad3)
    dst = jnp.pad(edge_index[1].reshape(NW, NCHUNK, CH), pad3)

    _sc_agg = _get_sc_agg()
    parts_x = _sc_agg(x, src, dst)

    w1a0 = jnp.pad(p['ga_W1_0'], ((0, 1), (0, 0)))

    def r2(v):
        return v.reshape(1, -1)

    hd, ha = _l0(
        x, parts_x, parts_x,
        p['gd_W1_0'], r2(p['gd_b1_0']), p['gd_W2_0'], r2(p['gd_b2_0']),
        w1a0, r2(p['ga_b1_0']), p['ga_W2_0'], r2(p['ga_b2_0']),
    )

    src2 = jnp.pad(edge_index[0].reshape(NS, NCHUNK2, CH),
                   ((0, 0), (0, NBLKI2 * CPB - NCHUNK2), (0, 0)))
    dst2 = jnp.pad(edge_index[1].reshape(NS, NCHUNK2, CH),
                   ((0, 0), (0, NBLKI2 * CPB - NCHUNK2), (0, 0)))
    parts2 = _get_sc_agg2()(hd, ha, src2, dst2)

    batch3 = batch.reshape(NBLK, 1, BLK)
    alpha = jax.nn.sigmoid(p['alpha']).reshape(1)
    out = _l1(
        hd, parts2, ha, parts2, batch3,
        p['gd_W1_1'], r2(p['gd_b1_1']), p['gd_W2_1'], r2(p['gd_b2_1']),
        p['ga_W1_1'], r2(p['ga_b1_1']), p['ga_W2_1'], r2(p['ga_b2_1']),
        p['gd_Wo'], r2(p['gd_bo']), p['ga_Wo'], r2(p['ga_bo']),
        alpha,
    )
    return out


# speedup vs baseline: 1.1464x; 1.0253x over previous
import functools

import jax
import jax.numpy as jnp
from jax import lax
from jax.experimental import pallas as pl
from jax.experimental.pallas import tpu as pltpu
from jax.experimental.pallas import tpu_sc as plsc

N = 10000
E = 320000
FEA = 128
HID = 128
TGT = 16
G = 64

NC = 2
NS = 16
NW = NC * NS
EPW = E // NW
CH = 80
NCHUNK = EPW // CH
CPB = 24
NBLKI = 6
NPAD = 10240
RPW = NPAD // NS
ZR = 8


def _zero_issue(sid, zero_v, agg_sh, sem):
    z16 = jnp.zeros((16,), jnp.float32)

    def zrow(i, carry):
        for j in range(FEA // 16):
            zero_v[i, pl.ds(j * 16, 16)] = z16
        return carry

    lax.fori_loop(0, ZR, zrow, 0)

    def zcopy(i, carry):
        pltpu.async_copy(zero_v, agg_sh.at[pl.ds(sid * RPW + i * ZR, ZR)],
                         sem)
        return carry

    lax.fori_loop(0, RPW // ZR, zcopy, 0)


def _zero_drain(sid, zero_v, agg_sh, sem):
    def zdrain(i, carry):
        pltpu.make_async_copy(
            zero_v, agg_sh.at[pl.ds(sid * RPW + i * ZR, ZR)], sem).wait()
        return carry

    lax.fori_loop(0, RPW // ZR, zdrain, 0)


def _writeout(cid, sid, agg_sh, out_hbm):
    @pl.when(sid < NS - 1)
    def _():
        pltpu.sync_copy(agg_sh.at[pl.ds(sid * RPW, RPW)],
                        out_hbm.at[pl.ds(cid * N + sid * RPW, RPW)])

    @pl.when(sid == NS - 1)
    def _():
        pltpu.sync_copy(agg_sh.at[pl.ds((NS - 1) * RPW, N - (NS - 1) * RPW)],
                        out_hbm.at[pl.ds(cid * N + (NS - 1) * RPW,
                                         N - (NS - 1) * RPW)])


def _stage_and_prime(h_hbm, e_hbm, row, b, src_v, dst_v, bufs, semg):
    pltpu.sync_copy(e_hbm.at[0, row, pl.ds(b * CPB, CPB)], src_v)
    pltpu.sync_copy(e_hbm.at[1, row, pl.ds(b * CPB, CPB)], dst_v)
    pltpu.async_copy(h_hbm.at[src_v.at[0]], bufs[0], semg[0])
    pltpu.async_copy(h_hbm.at[src_v.at[1]], bufs[1], semg[1])


def _edge_pipeline(h_hbm, e_hbm, row, nchunk, nblki,
                   src_v, dst_v, bufs, semg, sems, agg_sh, primed):
    def block(b, carry):
        base_c = b * CPB

        if primed:
            @pl.when(b > 0)
            def _():
                _stage_and_prime(h_hbm, e_hbm, row, b, src_v, dst_v,
                                 bufs, semg)
        else:
            _stage_and_prime(h_hbm, e_hbm, row, b, src_v, dst_v,
                             bufs, semg)

        def group(g, carry2):
            for j in range(3):
                c = 3 * g + j
                cp = c + 2
                jp = (j + 2) % 3

                @pl.when((cp < CPB) & (cp >= 3)
                         & (base_c + cp - 3 < nchunk))
                def _():
                    pltpu.make_async_copy(
                        bufs[jp], agg_sh.at[dst_v.at[cp - 3]],
                        sems[jp]).wait()

                @pl.when((cp < CPB) & (base_c + cp < nchunk))
                def _():
                    pltpu.async_copy(h_hbm.at[src_v.at[cp]], bufs[jp],
                                     semg[jp])

                @pl.when(base_c + c < nchunk)
                def _():
                    pltpu.make_async_copy(h_hbm.at[src_v.at[c]], bufs[j],
                                          semg[j]).wait()
                    pltpu.async_copy(bufs[j], agg_sh.at[dst_v.at[c]],
                                     sems[j], add=True)

            return carry2

        lax.fori_loop(0, CPB // 3, group, 0)

        for t in range(CPB - 3, CPB):
            @pl.when(base_c + t < nchunk)
            def _():
                pltpu.make_async_copy(bufs[t % 3],
                                      agg_sh.at[dst_v.at[t]],
                                      sems[t % 3]).wait()

        return carry

    lax.fori_loop(0, nblki, block, 0)


def _sc_agg_body(h_hbm, e_hbm, out_hbm,
                 src_v, dst_v, buf0_v, buf1_v, buf2_v, zero_v, agg_sh,
                 semg0, semg1, semg2, sems0, sems1, sems2):
    cid = lax.axis_index("c")
    sid = lax.axis_index("s")
    wid = cid * NS + sid
    bufs = (buf0_v, buf1_v, buf2_v)
    semg = (semg0, semg1, semg2)
    _zero_issue(sid, zero_v, agg_sh, sems0)
    _stage_and_prime(h_hbm, e_hbm, wid, 0, src_v, dst_v, bufs, semg)
    _zero_drain(sid, zero_v, agg_sh, sems0)
    plsc.subcore_barrier()
    _edge_pipeline(h_hbm, e_hbm, wid, NCHUNK, NBLKI,
                   src_v, dst_v, bufs, semg, (sems0, sems1, sems2), agg_sh,
                   primed=True)
    plsc.subcore_barrier()
    _writeout(cid, sid, agg_sh, out_hbm)


def _sc_agg2_body(hd_hbm, ha_hbm, e_hbm, out_hbm,
                  src_v, dst_v, buf0_v, buf1_v, buf2_v, zero_v, agg_sh,
                  semg0, semg1, semg2, sems0, sems1, sems2):
    cid = lax.axis_index("c")
    sid = lax.axis_index("s")
    bufs = (buf0_v, buf1_v, buf2_v)
    semg = (semg0, semg1, semg2)
    sems = (sems0, sems1, sems2)
    _zero_issue(sid, zero_v, agg_sh, sems0)

    @pl.when(cid == 0)
    def _():
        _stage_and_prime(hd_hbm, e_hbm, 2 * sid, 0, src_v, dst_v,
                         bufs, semg)

    @pl.when(cid == 1)
    def _():
        _stage_and_prime(ha_hbm, e_hbm, 2 * sid, 0, src_v, dst_v,
                         bufs, semg)

    _zero_drain(sid, zero_v, agg_sh, sems0)
    plsc.subcore_barrier()

    @pl.when(cid == 0)
    def _():
        _edge_pipeline(hd_hbm, e_hbm, 2 * sid, NCHUNK, NBLKI, src_v, dst_v,
                       bufs, semg, sems, agg_sh, primed=True)
        _edge_pipeline(hd_hbm, e_hbm, 2 * sid + 1, NCHUNK, NBLKI, src_v,
                       dst_v, bufs, semg, sems, agg_sh, primed=False)

    @pl.when(cid == 1)
    def _():
        _edge_pipeline(ha_hbm, e_hbm, 2 * sid, NCHUNK, NBLKI, src_v, dst_v,
                       bufs, semg, sems, agg_sh, primed=True)
        _edge_pipeline(ha_hbm, e_hbm, 2 * sid + 1, NCHUNK, NBLKI, src_v,
                       dst_v, bufs, semg, sems, agg_sh, primed=False)

    plsc.subcore_barrier()
    _writeout(cid, sid, agg_sh, out_hbm)


_SC_SCRATCH = [
    pltpu.VMEM((CPB, CH), jnp.int32),
    pltpu.VMEM((CPB, CH), jnp.int32),
    pltpu.VMEM((CH, FEA), jnp.float32),
    pltpu.VMEM((CH, FEA), jnp.float32),
    pltpu.VMEM((CH, FEA), jnp.float32),
    pltpu.VMEM((ZR, FEA), jnp.float32),
    pltpu.VMEM_SHARED((NPAD, FEA), jnp.float32),
    pltpu.SemaphoreType.DMA,
    pltpu.SemaphoreType.DMA,
    pltpu.SemaphoreType.DMA,
    pltpu.SemaphoreType.DMA,
    pltpu.SemaphoreType.DMA,
    pltpu.SemaphoreType.DMA,
]


@functools.cache
def _get_sc_agg():
    return pl.kernel(
        _sc_agg_body,
        mesh=plsc.VectorSubcoreMesh(core_axis_name="c", subcore_axis_name="s"),
        out_type=jax.ShapeDtypeStruct((NC * N, FEA), jnp.float32),
        scratch_types=list(_SC_SCRATCH),
    )


@functools.cache
def _get_sc_agg2():
    return pl.kernel(
        _sc_agg2_body,
        mesh=plsc.VectorSubcoreMesh(core_axis_name="c", subcore_axis_name="s"),
        out_type=jax.ShapeDtypeStruct((NC * N, FEA), jnp.float32),
        scratch_types=list(_SC_SCRATCH),
    )

BLK = 1000
NBLK = N // BLK
_PREC = jax.lax.Precision.DEFAULT


def _mm(a, b):
    return jnp.dot(a, b, precision=_PREC, preferred_element_type=jnp.float32)


def _l0_body(x_ref, p0_ref, p1_ref,
             w1d_ref, b1d_ref, w2d_ref, b2d_ref,
             w1a_ref, b1a_ref, w2a_ref, b2a_ref,
             hd_ref, ha_ref):
    m = x_ref[...] + p0_ref[...] + p1_ref[...]
    md = m[:, FEA - 1:FEA]
    td = jnp.maximum(md * w1d_ref[...] + b1d_ref[...], 0.0)
    hd_ref[...] = jnp.maximum(_mm(td, w2d_ref[...]) + b2d_ref[...], 0.0)
    ta = jnp.maximum(_mm(m[:, :FEA - 1], w1a_ref[...]) + b1a_ref[...], 0.0)
    ha_ref[...] = jnp.maximum(_mm(ta, w2a_ref[...]) + b2a_ref[...], 0.0)


def _full(shape):
    return pl.BlockSpec(shape, lambda i: (0, 0))


def _rows(shape):
    return pl.BlockSpec(shape, lambda i: (i, 0))


_l0 = pl.pallas_call(
    _l0_body,
    grid=(NBLK,),
    in_specs=[
        _rows((BLK, FEA)),
        _rows((BLK, FEA)),
        pl.BlockSpec((BLK, FEA), lambda i: (i + NBLK, 0)),
        _full((1, HID)), _full((1, HID)), _full((HID, HID)), _full((1, HID)),
        _full((FEA - 1, HID)), _full((1, HID)), _full((HID, HID)),
        _full((1, HID)),
    ],
    out_specs=[_rows((BLK, HID)), _rows((BLK, HID))],
    out_shape=[jax.ShapeDtypeStruct((N, HID), jnp.float32),
               jax.ShapeDtypeStruct((N, HID), jnp.float32)],
)


def _log_softmax(o):
    mx = jnp.max(o, axis=-1, keepdims=True)
    return o - (jnp.log(jnp.sum(jnp.exp(o - mx), axis=-1, keepdims=True)) + mx)


def _l1_body(hd_ref, pd_ref, ha_ref, pa_ref, b3_ref,
             w1d_ref, b1d_ref, w2d_ref, b2d_ref,
             w1a_ref, b1a_ref, w2a_ref, b2a_ref,
             wod_ref, bod_ref, woa_ref, boa_ref, al_ref,
             out_ref, poold_ref, poola_ref):
    i = pl.program_id(0)
    md = hd_ref[...] + pd_ref[...]
    td = jnp.maximum(_mm(md, w1d_ref[...]) + b1d_ref[...], 0.0)
    h2d = jnp.maximum(_mm(td, w2d_ref[...]) + b2d_ref[...], 0.0)
    ma = ha_ref[...] + pa_ref[...]
    ta = jnp.maximum(_mm(ma, w1a_ref[...]) + b1a_ref[...], 0.0)
    h2a = jnp.maximum(_mm(ta, w2a_ref[...]) + b2a_ref[...], 0.0)

    bvec = b3_ref[0]
    oh = (lax.broadcasted_iota(jnp.int32, (G, BLK), 0)
          == jnp.broadcast_to(bvec, (G, BLK))).astype(jnp.float32)

    @pl.when(i == 0)
    def _():
        poold_ref[...] = jnp.zeros_like(poold_ref)
        poola_ref[...] = jnp.zeros_like(poola_ref)

    poold_ref[...] += _mm(oh, h2d)
    poola_ref[...] += _mm(oh, h2a)

    @pl.when(i == NBLK - 1)
    def _():
        o1 = _mm(poold_ref[...], wod_ref[...]) + bod_ref[...]
        o2 = _mm(poola_ref[...], woa_ref[...]) + boa_ref[...]
        a = 1.0 / (1.0 + jnp.exp(-al_ref[0]))
        out_ref[...] = a * _log_softmax(o1) + (1.0 - a) * _log_softmax(o2)


_l1 = pl.pallas_call(
    _l1_body,
    grid=(NBLK,),
    in_specs=[
        _rows((BLK, HID)),
        _rows((BLK, HID)),
        _rows((BLK, HID)),
        pl.BlockSpec((BLK, HID), lambda i: (i + NBLK, 0)),
        pl.BlockSpec((1, 1, BLK), lambda i: (i, 0, 0)),
        _full((HID, HID)), _full((1, HID)), _full((HID, HID)), _full((1, HID)),
        _full((HID, HID)), _full((1, HID)), _full((HID, HID)), _full((1, HID)),
        _full((HID, TGT)), _full((1, TGT)), _full((HID, TGT)),
        _full((1, TGT)),
        pl.BlockSpec(memory_space=pltpu.SMEM),
    ],
    out_specs=pl.BlockSpec((G, TGT), lambda i: (0, 0)),
    out_shape=jax.ShapeDtypeStruct((G, TGT), jnp.float32),
    scratch_shapes=[pltpu.VMEM((G, HID), jnp.float32),
                    pltpu.VMEM((G, HID), jnp.float32)],
)


def kernel(x, edge_index, batch, params):
    p = params
    e3 = jnp.pad(edge_index.reshape(2, NW, NCHUNK, CH),
                 ((0, 0), (0, 0), (0, NBLKI * CPB - NCHUNK), (0, 0)))

    parts_x = _get_sc_agg()(x, e3)

    def r2(v):
        return v.reshape(1, -1)

    hd, ha = _l0(
        x, parts_x, parts_x,
        p['gd_W1_0'], r2(p['gd_b1_0']), p['gd_W2_0'], r2(p['gd_b2_0']),
        p['ga_W1_0'], r2(p['ga_b1_0']), p['ga_W2_0'], r2(p['ga_b2_0']),
    )

    parts2 = _get_sc_agg2()(hd, ha, e3)

    batch3 = batch.reshape(NBLK, 1, BLK)
    alpha = p['alpha'].reshape(1)
    out = _l1(
        hd, parts2, ha, parts2, batch3,
        p['gd_W1_1'], r2(p['gd_b1_1']), p['gd_W2_1'], r2(p['gd_b2_1']),
        p['ga_W1_1'], r2(p['ga_b1_1']), p['ga_W2_1'], r2(p['ga_b2_1']),
        p['gd_Wo'], r2(p['gd_bo']), p['ga_Wo'], r2(p['ga_bo']),
        alpha,
    )
    return out


# speedup vs baseline: 1.1675x; 1.0184x over previous
import functools

import jax
import jax.numpy as jnp
from jax import lax
from jax.experimental import pallas as pl
from jax.experimental.pallas import tpu as pltpu
from jax.experimental.pallas import tpu_sc as plsc

N = 10000
E = 320000
FEA = 128
HID = 128
TGT = 16
G = 64

NC = 2
NS = 16
NW = NC * NS
EPW = E // NW
CH = 80
NCHUNK = EPW // CH
CPB = 24
NBLKI = 6
NPAD = 10240
RPW = NPAD // NS
ZR = 8


def _zero_issue(sid, zero_v, agg_sh, sem):
    z16 = jnp.zeros((16,), jnp.float32)

    def zrow(i, carry):
        for j in range(FEA // 16):
            zero_v[i, pl.ds(j * 16, 16)] = z16
        return carry

    lax.fori_loop(0, ZR, zrow, 0)

    def zcopy(i, carry):
        pltpu.async_copy(zero_v, agg_sh.at[pl.ds(sid * RPW + i * ZR, ZR)],
                         sem)
        return carry

    lax.fori_loop(0, RPW // ZR, zcopy, 0)


def _zero_drain(sid, zero_v, agg_sh, sem):
    def zdrain(i, carry):
        pltpu.make_async_copy(
            zero_v, agg_sh.at[pl.ds(sid * RPW + i * ZR, ZR)], sem).wait()
        return carry

    lax.fori_loop(0, RPW // ZR, zdrain, 0)


def _writeout(cid, sid, agg_sh, out_hbm):
    @pl.when(sid < NS - 1)
    def _():
        pltpu.sync_copy(agg_sh.at[pl.ds(sid * RPW, RPW)],
                        out_hbm.at[pl.ds(cid * N + sid * RPW, RPW)])

    @pl.when(sid == NS - 1)
    def _():
        pltpu.sync_copy(agg_sh.at[pl.ds((NS - 1) * RPW, N - (NS - 1) * RPW)],
                        out_hbm.at[pl.ds(cid * N + (NS - 1) * RPW,
                                         N - (NS - 1) * RPW)])


def _stage_and_prime(h_hbm, e_hbm, row, b, src_v, dst_v, bufs, semg):
    pltpu.sync_copy(e_hbm.at[0, row, pl.ds(b * CPB, CPB)], src_v)
    pltpu.sync_copy(e_hbm.at[1, row, pl.ds(b * CPB, CPB)], dst_v)
    pltpu.async_copy(h_hbm.at[src_v.at[0]], bufs[0], semg[0])
    pltpu.async_copy(h_hbm.at[src_v.at[1]], bufs[1], semg[1])


def _edge_pipeline(h_hbm, e_hbm, row, nchunk, nblki,
                   src_v, dst_v, bufs, semg, sems, agg_sh, primed):
    def block(b, carry):
        base_c = b * CPB

        if primed:
            @pl.when(b > 0)
            def _():
                _stage_and_prime(h_hbm, e_hbm, row, b, src_v, dst_v,
                                 bufs, semg)
        else:
            _stage_and_prime(h_hbm, e_hbm, row, b, src_v, dst_v,
                             bufs, semg)

        def group(g, carry2):
            for j in range(3):
                c = 3 * g + j
                cp = c + 2
                jp = (j + 2) % 3

                @pl.when((cp < CPB) & (cp >= 3)
                         & (base_c + cp - 3 < nchunk))
                def _():
                    pltpu.make_async_copy(
                        bufs[jp], agg_sh.at[dst_v.at[cp - 3]],
                        sems[jp]).wait()

                @pl.when((cp < CPB) & (base_c + cp < nchunk))
                def _():
                    pltpu.async_copy(h_hbm.at[src_v.at[cp]], bufs[jp],
                                     semg[jp])

                @pl.when(base_c + c < nchunk)
                def _():
                    pltpu.make_async_copy(h_hbm.at[src_v.at[c]], bufs[j],
                                          semg[j]).wait()
                    pltpu.async_copy(bufs[j], agg_sh.at[dst_v.at[c]],
                                     sems[j], add=True)

            return carry2

        lax.fori_loop(0, CPB // 3, group, 0)

        for t in range(CPB - 3, CPB):
            @pl.when(base_c + t < nchunk)
            def _():
                pltpu.make_async_copy(bufs[t % 3],
                                      agg_sh.at[dst_v.at[t]],
                                      sems[t % 3]).wait()

        return carry

    lax.fori_loop(0, nblki, block, 0)


def _sc_agg_body(h_hbm, e_hbm, out_hbm,
                 src_v, dst_v, buf0_v, buf1_v, buf2_v, zero_v, agg_sh,
                 semg0, semg1, semg2, sems0, sems1, sems2):
    cid = lax.axis_index("c")
    sid = lax.axis_index("s")
    wid = cid * NS + sid
    bufs = (buf0_v, buf1_v, buf2_v)
    semg = (semg0, semg1, semg2)
    _zero_issue(sid, zero_v, agg_sh, sems0)
    _stage_and_prime(h_hbm, e_hbm, wid, 0, src_v, dst_v, bufs, semg)
    _zero_drain(sid, zero_v, agg_sh, sems0)
    plsc.subcore_barrier()
    _edge_pipeline(h_hbm, e_hbm, wid, NCHUNK, NBLKI,
                   src_v, dst_v, bufs, semg, (sems0, sems1, sems2), agg_sh,
                   primed=True)
    plsc.subcore_barrier()
    _writeout(cid, sid, agg_sh, out_hbm)


def _sc_agg2_body(hd_hbm, ha_hbm, e_hbm, out_hbm,
                  src_v, dst_v, buf0_v, buf1_v, buf2_v, zero_v, agg_sh,
                  semg0, semg1, semg2, sems0, sems1, sems2):
    cid = lax.axis_index("c")
    sid = lax.axis_index("s")
    bufs = (buf0_v, buf1_v, buf2_v)
    semg = (semg0, semg1, semg2)
    sems = (sems0, sems1, sems2)
    _zero_issue(sid, zero_v, agg_sh, sems0)

    @pl.when(cid == 0)
    def _():
        _stage_and_prime(hd_hbm, e_hbm, 2 * sid, 0, src_v, dst_v,
                         bufs, semg)

    @pl.when(cid == 1)
    def _():
        _stage_and_prime(ha_hbm, e_hbm, 2 * sid, 0, src_v, dst_v,
                         bufs, semg)

    _zero_drain(sid, zero_v, agg_sh, sems0)
    plsc.subcore_barrier()

    @pl.when(cid == 0)
    def _():
        _edge_pipeline(hd_hbm, e_hbm, 2 * sid, NCHUNK, NBLKI, src_v, dst_v,
                       bufs, semg, sems, agg_sh, primed=True)
        _edge_pipeline(hd_hbm, e_hbm, 2 * sid + 1, NCHUNK, NBLKI, src_v,
                       dst_v, bufs, semg, sems, agg_sh, primed=False)

    @pl.when(cid == 1)
    def _():
        _edge_pipeline(ha_hbm, e_hbm, 2 * sid, NCHUNK, NBLKI, src_v, dst_v,
                       bufs, semg, sems, agg_sh, primed=True)
        _edge_pipeline(ha_hbm, e_hbm, 2 * sid + 1, NCHUNK, NBLKI, src_v,
                       dst_v, bufs, semg, sems, agg_sh, primed=False)

    plsc.subcore_barrier()
    _writeout(cid, sid, agg_sh, out_hbm)


_SC_SCRATCH = [
    pltpu.VMEM((CPB, CH), jnp.int32),
    pltpu.VMEM((CPB, CH), jnp.int32),
    pltpu.VMEM((CH, FEA), jnp.float32),
    pltpu.VMEM((CH, FEA), jnp.float32),
    pltpu.VMEM((CH, FEA), jnp.float32),
    pltpu.VMEM((ZR, FEA), jnp.float32),
    pltpu.VMEM_SHARED((NPAD, FEA), jnp.float32),
    pltpu.SemaphoreType.DMA,
    pltpu.SemaphoreType.DMA,
    pltpu.SemaphoreType.DMA,
    pltpu.SemaphoreType.DMA,
    pltpu.SemaphoreType.DMA,
    pltpu.SemaphoreType.DMA,
]


@functools.cache
def _get_sc_agg():
    return pl.kernel(
        _sc_agg_body,
        mesh=plsc.VectorSubcoreMesh(core_axis_name="c", subcore_axis_name="s"),
        out_type=jax.ShapeDtypeStruct((NC * N, FEA), jnp.float32),
        scratch_types=list(_SC_SCRATCH),
    )


@functools.cache
def _get_sc_agg2():
    return pl.kernel(
        _sc_agg2_body,
        mesh=plsc.VectorSubcoreMesh(core_axis_name="c", subcore_axis_name="s"),
        out_type=jax.ShapeDtypeStruct((NC * N, FEA), jnp.float32),
        scratch_types=list(_SC_SCRATCH),
    )

BLK = 2000
NBLK = N // BLK
_PREC = jax.lax.Precision.DEFAULT


def _mm(a, b):
    return jnp.dot(a, b, precision=_PREC, preferred_element_type=jnp.float32)


def _l0_body(x_ref, p0_ref, p1_ref,
             w1d_ref, b1d_ref, w2d_ref, b2d_ref,
             w1a_ref, b1a_ref, w2a_ref, b2a_ref,
             hd_ref, ha_ref):
    m = x_ref[...] + p0_ref[...] + p1_ref[...]
    md = m[:, FEA - 1:FEA]
    td = jnp.maximum(md * w1d_ref[...] + b1d_ref[...], 0.0)
    hd_ref[...] = jnp.maximum(_mm(td, w2d_ref[...]) + b2d_ref[...], 0.0)
    ta = jnp.maximum(_mm(m[:, :FEA - 1], w1a_ref[...]) + b1a_ref[...], 0.0)
    ha_ref[...] = jnp.maximum(_mm(ta, w2a_ref[...]) + b2a_ref[...], 0.0)


def _full(shape):
    return pl.BlockSpec(shape, lambda i: (0, 0))


def _rows(shape):
    return pl.BlockSpec(shape, lambda i: (i, 0))


_l0 = pl.pallas_call(
    _l0_body,
    grid=(NBLK,),
    in_specs=[
        _rows((BLK, FEA)),
        _rows((BLK, FEA)),
        pl.BlockSpec((BLK, FEA), lambda i: (i + NBLK, 0)),
        _full((1, HID)), _full((1, HID)), _full((HID, HID)), _full((1, HID)),
        _full((FEA - 1, HID)), _full((1, HID)), _full((HID, HID)),
        _full((1, HID)),
    ],
    out_specs=[_rows((BLK, HID)), _rows((BLK, HID))],
    out_shape=[jax.ShapeDtypeStruct((N, HID), jnp.float32),
               jax.ShapeDtypeStruct((N, HID), jnp.float32)],
)


def _log_softmax(o):
    mx = jnp.max(o, axis=-1, keepdims=True)
    return o - (jnp.log(jnp.sum(jnp.exp(o - mx), axis=-1, keepdims=True)) + mx)


def _l1_body(hd_ref, pd_ref, ha_ref, pa_ref, b3_ref,
             w1d_ref, b1d_ref, w2d_ref, b2d_ref,
             w1a_ref, b1a_ref, w2a_ref, b2a_ref,
             wod_ref, bod_ref, woa_ref, boa_ref, al_ref,
             out_ref, poold_ref, poola_ref):
    i = pl.program_id(0)
    md = hd_ref[...] + pd_ref[...]
    td = jnp.maximum(_mm(md, w1d_ref[...]) + b1d_ref[...], 0.0)
    h2d = jnp.maximum(_mm(td, w2d_ref[...]) + b2d_ref[...], 0.0)
    ma = ha_ref[...] + pa_ref[...]
    ta = jnp.maximum(_mm(ma, w1a_ref[...]) + b1a_ref[...], 0.0)
    h2a = jnp.maximum(_mm(ta, w2a_ref[...]) + b2a_ref[...], 0.0)

    bvec = b3_ref[0]
    oh = (lax.broadcasted_iota(jnp.int32, (G, BLK), 0)
          == jnp.broadcast_to(bvec, (G, BLK))).astype(jnp.float32)

    @pl.when(i == 0)
    def _():
        poold_ref[...] = jnp.zeros_like(poold_ref)
        poola_ref[...] = jnp.zeros_like(poola_ref)

    poold_ref[...] += _mm(oh, h2d)
    poola_ref[...] += _mm(oh, h2a)

    @pl.when(i == NBLK - 1)
    def _():
        o1 = _mm(poold_ref[...], wod_ref[...]) + bod_ref[...]
        o2 = _mm(poola_ref[...], woa_ref[...]) + boa_ref[...]
        a = 1.0 / (1.0 + jnp.exp(-al_ref[0]))
        out_ref[...] = a * _log_softmax(o1) + (1.0 - a) * _log_softmax(o2)


_l1 = pl.pallas_call(
    _l1_body,
    grid=(NBLK,),
    in_specs=[
        _rows((BLK, HID)),
        _rows((BLK, HID)),
        _rows((BLK, HID)),
        pl.BlockSpec((BLK, HID), lambda i: (i + NBLK, 0)),
        pl.BlockSpec((1, 1, BLK), lambda i: (i, 0, 0)),
        _full((HID, HID)), _full((1, HID)), _full((HID, HID)), _full((1, HID)),
        _full((HID, HID)), _full((1, HID)), _full((HID, HID)), _full((1, HID)),
        _full((HID, TGT)), _full((1, TGT)), _full((HID, TGT)),
        _full((1, TGT)),
        pl.BlockSpec(memory_space=pltpu.SMEM),
    ],
    out_specs=pl.BlockSpec((G, TGT), lambda i: (0, 0)),
    out_shape=jax.ShapeDtypeStruct((G, TGT), jnp.float32),
    scratch_shapes=[pltpu.VMEM((G, HID), jnp.float32),
                    pltpu.VMEM((G, HID), jnp.float32)],
)


def kernel(x, edge_index, batch, params):
    p = params
    e3 = jnp.pad(edge_index.reshape(2, NW, NCHUNK, CH),
                 ((0, 0), (0, 0), (0, NBLKI * CPB - NCHUNK), (0, 0)))

    parts_x = _get_sc_agg()(x, e3)

    def r2(v):
        return v.reshape(1, -1)

    hd, ha = _l0(
        x, parts_x, parts_x,
        p['gd_W1_0'], r2(p['gd_b1_0']), p['gd_W2_0'], r2(p['gd_b2_0']),
        p['ga_W1_0'], r2(p['ga_b1_0']), p['ga_W2_0'], r2(p['ga_b2_0']),
    )

    parts2 = _get_sc_agg2()(hd, ha, e3)

    batch3 = batch.reshape(NBLK, 1, BLK)
    alpha = p['alpha'].reshape(1)
    out = _l1(
        hd, parts2, ha, parts2, batch3,
        p['gd_W1_1'], r2(p['gd_b1_1']), p['gd_W2_1'], r2(p['gd_b2_1']),
        p['ga_W1_1'], r2(p['ga_b1_1']), p['ga_W2_1'], r2(p['ga_b2_1']),
        p['gd_Wo'], r2(p['gd_bo']), p['ga_Wo'], r2(p['ga_bo']),
        alpha,
    )
    return out
